# Initial kernel scaffold; baseline (speedup 1.0000x reference)
import jax
import jax.numpy as jnp
from jax.experimental import pallas as pl


def kernel(nodes, neigh_idx, num_sample, table):
    raise NotImplementedError("write your pallas kernel here")


# speedup vs baseline: 3.4387x; 3.4387x over previous
import functools

import jax
import jax.numpy as jnp
from jax import lax
from jax.experimental import pallas as pl
from jax.experimental.pallas import tpu as pltpu
from jax.experimental.pallas import tpu_sc as plsc

B = 16384
D = 128
S = 25
L = 16
NC = 2
NS = 16
NW = NC * NS
ROWS_PER_W = B // NW
R = 4
IDX_PER_STEP = R * S
NSTEPS = ROWS_PER_W // R


def _make_sc_call():
    mesh = plsc.VectorSubcoreMesh(core_axis_name="c", subcore_axis_name="s")

    @functools.partial(
        pl.kernel,
        mesh=mesh,
        out_type=jax.ShapeDtypeStruct((B, D), jnp.float32),
        scratch_types=[
            pltpu.VMEM((NSTEPS, IDX_PER_STEP), jnp.int32),
            pltpu.VMEM((IDX_PER_STEP, D), jnp.float32),
            pltpu.VMEM((ROWS_PER_W, D), jnp.float32),
            pltpu.SemaphoreType.DMA,
        ],
    )
    def sc_mean(table_hbm, idx_hbm, out_hbm, idx_v, buf, out_v, sem):
        wid = lax.axis_index("s") * NC + lax.axis_index("c")
        pltpu.sync_copy(idx_hbm.at[wid], idx_v)

        def step(g, carry):
            pltpu.async_copy(table_hbm.at[idx_v.at[g]], buf, sem).wait()
            for r in range(R):
                for c in range(D // L):
                    acc = buf[r * S, pl.ds(c * L, L)]
                    for s in range(1, S):
                        acc = acc + buf[r * S + s, pl.ds(c * L, L)]
                    out_v[g * R + r, pl.ds(c * L, L)] = acc * (1.0 / S)
            return carry

        lax.fori_loop(0, NSTEPS, step, 0)
        pltpu.sync_copy(out_v, out_hbm.at[pl.ds(wid * ROWS_PER_W, ROWS_PER_W)])

    return sc_mean


_sc_mean = _make_sc_call()


def kernel(nodes, neigh_idx, num_sample, table):
    del nodes, num_sample
    idx = jnp.reshape(neigh_idx.astype(jnp.int32), (NW, NSTEPS, IDX_PER_STEP))
    return _sc_mean(table, idx)


# speedup vs baseline: 4.4784x; 1.3024x over previous
import functools

import jax
import jax.numpy as jnp
from jax import lax
from jax.experimental import pallas as pl
from jax.experimental.pallas import tpu as pltpu
from jax.experimental.pallas import tpu_sc as plsc

B = 16384
D = 128
S = 25
L = 16
NC = 2
NS = 16
NW = NC * NS
ROWS_PER_W = B // NW
R = 4
IDX_PER_STEP = R * S
NSTEPS = ROWS_PER_W // R


def _make_sc_call():
    mesh = plsc.VectorSubcoreMesh(core_axis_name="c", subcore_axis_name="s")

    @functools.partial(
        pl.kernel,
        mesh=mesh,
        out_type=jax.ShapeDtypeStruct((B, D), jnp.float32),
        scratch_types=[
            pltpu.VMEM((NSTEPS, IDX_PER_STEP), jnp.int32),
            pltpu.VMEM((IDX_PER_STEP, D), jnp.float32),
            pltpu.VMEM((IDX_PER_STEP, D), jnp.float32),
            pltpu.VMEM((ROWS_PER_W, D), jnp.float32),
            pltpu.SemaphoreType.DMA,
            pltpu.SemaphoreType.DMA,
        ],
    )
    def sc_mean(table_hbm, idx_hbm, out_hbm, idx_v, buf0, buf1, out_v, sem0, sem1):
        wid = lax.axis_index("s") * NC + lax.axis_index("c")
        pltpu.sync_copy(idx_hbm.at[wid], idx_v)

        def start(g, buf, sem):
            pltpu.async_copy(table_hbm.at[idx_v.at[g]], buf, sem)

        def drain(g, buf, sem):
            pltpu.make_async_copy(table_hbm.at[idx_v.at[g]], buf, sem).wait()

        def compute(g, buf):
            for r in range(R):
                for c in range(D // L):
                    acc = buf[r * S, pl.ds(c * L, L)]
                    for s in range(1, S):
                        acc = acc + buf[r * S + s, pl.ds(c * L, L)]
                    out_v[g * R + r, pl.ds(c * L, L)] = acc * (1.0 / S)

        start(0, buf0, sem0)

        def step(i, carry):
            g = i * 2
            start(g + 1, buf1, sem1)
            drain(g, buf0, sem0)
            compute(g, buf0)

            @pl.when(g + 2 < NSTEPS)
            def _():
                start(g + 2, buf0, sem0)

            drain(g + 1, buf1, sem1)
            compute(g + 1, buf1)
            return carry

        lax.fori_loop(0, NSTEPS // 2, step, 0)
        pltpu.sync_copy(out_v, out_hbm.at[pl.ds(wid * ROWS_PER_W, ROWS_PER_W)])

    return sc_mean


_sc_mean = _make_sc_call()


def kernel(nodes, neigh_idx, num_sample, table):
    del nodes, num_sample
    idx = jnp.reshape(neigh_idx.astype(jnp.int32), (NW, NSTEPS, IDX_PER_STEP))
    return _sc_mean(table, idx)


# speedup vs baseline: 6.0824x; 1.3582x over previous
import functools

import jax
import jax.numpy as jnp
from jax import lax
from jax.experimental import pallas as pl
from jax.experimental.pallas import tpu as pltpu
from jax.experimental.pallas import tpu_sc as plsc

B = 16384
D = 128
S = 25
L = 16
NC = 2
NS = 16
NW = NC * NS
ROWS_PER_W = B // NW
R = 4
IDX_PER_STEP = R * S
NSTEPS = ROWS_PER_W // R


def _make_sc_call():
    mesh = plsc.VectorSubcoreMesh(core_axis_name="c", subcore_axis_name="s")

    @functools.partial(
        pl.kernel,
        mesh=mesh,
        out_type=jax.ShapeDtypeStruct((B, D), jnp.float32),
        scratch_types=[
            pltpu.VMEM((NSTEPS, IDX_PER_STEP), jnp.int32),
            pltpu.VMEM((IDX_PER_STEP, D), jnp.float32),
            pltpu.VMEM((IDX_PER_STEP, D), jnp.float32),
            pltpu.VMEM((ROWS_PER_W, D), jnp.float32),
            pltpu.SemaphoreType.DMA,
            pltpu.SemaphoreType.DMA,
        ],
    )
    def sc_mean(table_hbm, idx_hbm, out_hbm, idx_v, buf0, buf1, out_v, sem0, sem1):
        wid = lax.axis_index("s") * NC + lax.axis_index("c")
        pltpu.sync_copy(idx_hbm.at[wid], idx_v)

        def start(g, buf, sem):
            pltpu.async_copy(table_hbm.at[idx_v.at[g]], buf, sem)

        def drain(g, buf, sem):
            pltpu.make_async_copy(table_hbm.at[idx_v.at[g]], buf, sem).wait()

        def compute(g, buf):
            for r in range(R):
                for c in range(D // L):
                    vals = [buf[r * S + s, pl.ds(c * L, L)] for s in range(S)]
                    while len(vals) > 1:
                        nxt = [a + b for a, b in zip(vals[0::2], vals[1::2])]
                        if len(vals) % 2:
                            nxt.append(vals[-1])
                        vals = nxt
                    out_v[g * R + r, pl.ds(c * L, L)] = vals[0] * (1.0 / S)

        start(0, buf0, sem0)

        def step(i, carry):
            g = i * 2
            start(g + 1, buf1, sem1)
            drain(g, buf0, sem0)
            compute(g, buf0)

            @pl.when(g + 2 < NSTEPS)
            def _():
                start(g + 2, buf0, sem0)

            drain(g + 1, buf1, sem1)
            compute(g + 1, buf1)
            return carry

        lax.fori_loop(0, NSTEPS // 2, step, 0)
        pltpu.sync_copy(out_v, out_hbm.at[pl.ds(wid * ROWS_PER_W, ROWS_PER_W)])

    return sc_mean


_sc_mean = _make_sc_call()


def kernel(nodes, neigh_idx, num_sample, table):
    del nodes, num_sample
    idx = jnp.reshape(neigh_idx.astype(jnp.int32), (NW, NSTEPS, IDX_PER_STEP))
    return _sc_mean(table, idx)


# speedup vs baseline: 8.3448x; 1.3720x over previous
import functools

import jax
import jax.numpy as jnp
from jax import lax
from jax.experimental import pallas as pl
from jax.experimental.pallas import tpu as pltpu
from jax.experimental.pallas import tpu_sc as plsc

B = 16384
D = 128
S = 25
L = 16
NC = 2
NS = 16
NW = NC * NS
ROWS_PER_W = B // NW
R = 4
IDX_PER_STEP = R * S
NSTEPS = ROWS_PER_W // R


def _make_sc_call():
    mesh = plsc.VectorSubcoreMesh(core_axis_name="c", subcore_axis_name="s")

    @functools.partial(
        pl.kernel,
        mesh=mesh,
        out_type=jax.ShapeDtypeStruct((B, D), jnp.float32),
        scratch_types=[
            pltpu.VMEM((NSTEPS, IDX_PER_STEP), jnp.int32),
            pltpu.VMEM((IDX_PER_STEP, D), jnp.float32),
            pltpu.VMEM((IDX_PER_STEP, D), jnp.float32),
            pltpu.VMEM((ROWS_PER_W, D), jnp.float32),
            pltpu.SemaphoreType.DMA,
            pltpu.SemaphoreType.DMA,
        ],
    )
    def sc_mean(table_hbm, idx_hbm, out_hbm, idx_v, buf0, buf1, out_v, sem0, sem1):
        wid = lax.axis_index("s") * NC + lax.axis_index("c")
        pltpu.sync_copy(idx_hbm.at[wid], idx_v)

        def start(g, buf, sem):
            pltpu.async_copy(table_hbm.at[idx_v.at[g]], buf, sem)

        def drain(g, buf, sem):
            pltpu.make_async_copy(table_hbm.at[idx_v.at[g]], buf, sem).wait()

        def compute(g, buf):
            def reduce_store(r, c, vals):
                while len(vals) > 1:
                    nxt = [a + b for a, b in zip(vals[0::2], vals[1::2])]
                    if len(vals) % 2:
                        nxt.append(vals[-1])
                    vals = nxt
                out_v[g * R + r, pl.ds(c * L, L)] = vals[0] * (1.0 / S)

            prev = None
            for r in range(R):
                for c in range(D // L):
                    vals = [buf[r * S + s, pl.ds(c * L, L)] for s in range(S)]
                    if prev is not None:
                        reduce_store(*prev)
                    prev = (r, c, vals)
            reduce_store(*prev)

        start(0, buf0, sem0)

        def step(i, carry):
            g = i * 2
            start(g + 1, buf1, sem1)
            drain(g, buf0, sem0)
            compute(g, buf0)

            @pl.when(g + 2 < NSTEPS)
            def _():
                start(g + 2, buf0, sem0)

            drain(g + 1, buf1, sem1)
            compute(g + 1, buf1)
            return carry

        lax.fori_loop(0, NSTEPS // 2, step, 0)
        pltpu.sync_copy(out_v, out_hbm.at[pl.ds(wid * ROWS_PER_W, ROWS_PER_W)])

    return sc_mean


_sc_mean = _make_sc_call()


def kernel(nodes, neigh_idx, num_sample, table):
    del nodes, num_sample
    idx = jnp.reshape(neigh_idx.astype(jnp.int32), (NW, NSTEPS, IDX_PER_STEP))
    return _sc_mean(table, idx)
